# baseline (speedup 1.0000x reference)
import functools

import jax
import jax.numpy as jnp
from jax.experimental import pallas as pl

_N = 8192
_D_IN = 2048
_D_OUT = 2048
_H1 = 100
_H2 = 50
_EPS = 1e-5
_BLOCK = 512


def _row_major_dot(a, b):
    return jax.lax.dot_general(
        a, b, (((1,), (1,)), ((), ())),
        preferred_element_type=jnp.float32,
        precision=jax.lax.Precision.DEFAULT)


def _mlp_block(x_ref, w1_ref, b1_ref, w2_ref, b2_ref, w3_ref, b3_ref,
               m0_ref, v0_ref, m1_ref, v1_ref, m2_ref, v2_ref, out_ref):
    s0 = jax.lax.rsqrt(v0_ref[...] + _EPS)
    s1 = jax.lax.rsqrt(v1_ref[...] + _EPS)
    s2 = jax.lax.rsqrt(v2_ref[...] + _EPS)

    w1e = w1_ref[...] * s0
    b1e = b1_ref[...] - _row_major_dot(m0_ref[...] * s0, w1_ref[...])
    w2e = w2_ref[...] * s1
    b2e = b2_ref[...] - _row_major_dot(m1_ref[...] * s1, w2_ref[...])
    w3e = w3_ref[...] * s2
    b3e = b3_ref[...] - _row_major_dot(m2_ref[...] * s2, w3_ref[...])

    h = jnp.maximum(_row_major_dot(x_ref[...], w1e) + b1e, 0.0)
    g = jnp.maximum(_row_major_dot(h, w2e) + b2e, 0.0)
    out_ref[...] = _row_major_dot(g, w3e) + b3e


@functools.partial(jax.jit, static_argnames=("interpret",))
def kernel(x, W1, b1, W2, b2, W3, b3, bn0_mean, bn0_var, bn1_mean, bn1_var,
           bn2_mean, bn2_var, interpret=False):
    n = x.shape[0]
    grid = (n // _BLOCK,)

    def row_blk(i):
        return (i, 0)

    def const_blk(i):
        return (0, 0)

    full = lambda shape: pl.BlockSpec(shape, const_blk)

    return pl.pallas_call(
        _mlp_block,
        grid=grid,
        in_specs=[
            pl.BlockSpec((_BLOCK, _D_IN), row_blk),
            full((_H1, _D_IN)),
            full((1, _H1)),
            full((_H2, _H1)),
            full((1, _H2)),
            full((_D_OUT, _H2)),
            full((1, _D_OUT)),
            full((1, _D_IN)),
            full((1, _D_IN)),
            full((1, _H1)),
            full((1, _H1)),
            full((1, _H2)),
            full((1, _H2)),
        ],
        out_specs=pl.BlockSpec((_BLOCK, _D_OUT), row_blk),
        out_shape=jax.ShapeDtypeStruct((n, _D_OUT), jnp.float32),
        interpret=interpret,
    )(x, W1, b1.reshape(1, -1), W2, b2.reshape(1, -1), W3,
      b3.reshape(1, -1), bn0_mean.reshape(1, -1), bn0_var.reshape(1, -1),
      bn1_mean.reshape(1, -1), bn1_var.reshape(1, -1),
      bn2_mean.reshape(1, -1), bn2_var.reshape(1, -1))


# speedup vs baseline: 1.0427x; 1.0427x over previous
import functools

import jax
import jax.numpy as jnp
from jax.experimental import pallas as pl
from jax.experimental.pallas import tpu as pltpu

_N = 8192
_D_IN = 2048
_D_OUT = 2048
_H1 = 100
_H2 = 50
_EPS = 1e-5
_BLOCK = 512


def _rm_dot(a, b):
    return jax.lax.dot_general(
        a, b, (((1,), (1,)), ((), ())),
        preferred_element_type=jnp.float32,
        precision=jax.lax.Precision.DEFAULT)


def _mlp_block(x_ref, w1_ref, b1_ref, w2_ref, b2_ref, w3_ref, b3_ref,
               m0_ref, v0_ref, m1_ref, v1_ref, m2_ref, v2_ref, out_ref,
               w1s, w2s, w3s, b1s, b2s, b3s):
    @pl.when(pl.program_id(0) == 0)
    def _fold():
        s0 = jax.lax.rsqrt(v0_ref[...] + _EPS)
        s1 = jax.lax.rsqrt(v1_ref[...] + _EPS)
        s2 = jax.lax.rsqrt(v2_ref[...] + _EPS)
        w1s[...] = (w1_ref[...] * s0).astype(jnp.bfloat16)
        w2s[...] = (w2_ref[...] * s1).astype(jnp.bfloat16)
        w3s[...] = (w3_ref[...] * s2).astype(jnp.bfloat16)
        b1s[...] = b1_ref[...] - _rm_dot(m0_ref[...] * s0, w1_ref[...])
        b2s[...] = b2_ref[...] - _rm_dot(m1_ref[...] * s1, w2_ref[...])
        b3s[...] = b3_ref[...] - _rm_dot(m2_ref[...] * s2, w3_ref[...])

    xb = x_ref[...].astype(jnp.bfloat16)
    h = jnp.maximum(_rm_dot(xb, w1s[...]) + b1s[...], 0.0)
    g = jnp.maximum(_rm_dot(h.astype(jnp.bfloat16), w2s[...]) + b2s[...], 0.0)
    out_ref[...] = _rm_dot(g.astype(jnp.bfloat16), w3s[...]) + b3s[...]


@functools.partial(jax.jit, static_argnames=("interpret",))
def kernel(x, W1, b1, W2, b2, W3, b3, bn0_mean, bn0_var, bn1_mean, bn1_var,
           bn2_mean, bn2_var, interpret=False):
    n = x.shape[0]
    grid = (n // _BLOCK,)

    def row_blk(i):
        return (i, 0)

    def const_blk(i):
        return (0, 0)

    full = lambda shape: pl.BlockSpec(shape, const_blk)

    return pl.pallas_call(
        _mlp_block,
        grid=grid,
        in_specs=[
            pl.BlockSpec((_BLOCK, _D_IN), row_blk),
            full((_H1, _D_IN)),
            full((1, _H1)),
            full((_H2, _H1)),
            full((1, _H2)),
            full((_D_OUT, _H2)),
            full((1, _D_OUT)),
            full((1, _D_IN)),
            full((1, _D_IN)),
            full((1, _H1)),
            full((1, _H1)),
            full((1, _H2)),
            full((1, _H2)),
        ],
        out_specs=pl.BlockSpec((_BLOCK, _D_OUT), row_blk),
        out_shape=jax.ShapeDtypeStruct((n, _D_OUT), jnp.float32),
        scratch_shapes=[
            pltpu.VMEM((_H1, _D_IN), jnp.bfloat16),
            pltpu.VMEM((_H2, _H1), jnp.bfloat16),
            pltpu.VMEM((_D_OUT, _H2), jnp.bfloat16),
            pltpu.VMEM((1, _H1), jnp.float32),
            pltpu.VMEM((1, _H2), jnp.float32),
            pltpu.VMEM((1, _D_OUT), jnp.float32),
        ],
        interpret=interpret,
    )(x, W1, b1.reshape(1, -1), W2, b2.reshape(1, -1), W3,
      b3.reshape(1, -1), bn0_mean.reshape(1, -1), bn0_var.reshape(1, -1),
      bn1_mean.reshape(1, -1), bn1_var.reshape(1, -1),
      bn2_mean.reshape(1, -1), bn2_var.reshape(1, -1))


# speedup vs baseline: 1.1329x; 1.0866x over previous
import functools

import jax
import jax.numpy as jnp
from jax.experimental import pallas as pl
from jax.experimental.pallas import tpu as pltpu

_N = 8192
_D_IN = 2048
_D_OUT = 2048
_H1 = 100
_H2 = 50
_EPS = 1e-5
_BLOCK = 1024


def _rm_dot(a, b):
    return jax.lax.dot_general(
        a, b, (((1,), (1,)), ((), ())),
        preferred_element_type=jnp.float32,
        precision=jax.lax.Precision.DEFAULT)


def _mlp_block(x_ref, w1_ref, b1_ref, w2_ref, b2_ref, w3_ref, b3_ref,
               m0_ref, v0_ref, m1_ref, v1_ref, m2_ref, v2_ref, out_ref,
               w1s, w2s, w3s, b1s, b2s, b3s):
    @pl.when(pl.program_id(0) == 0)
    def _fold():
        s0 = jax.lax.rsqrt(v0_ref[...] + _EPS)
        s1 = jax.lax.rsqrt(v1_ref[...] + _EPS)
        s2 = jax.lax.rsqrt(v2_ref[...] + _EPS)
        w1s[...] = (w1_ref[...] * s0).astype(jnp.bfloat16)
        w2s[...] = (w2_ref[...] * s1).astype(jnp.bfloat16)
        w3s[...] = (w3_ref[...] * s2).astype(jnp.bfloat16)
        b1s[...] = b1_ref[...] - _rm_dot(m0_ref[...] * s0, w1_ref[...])
        b2s[...] = b2_ref[...] - _rm_dot(m1_ref[...] * s1, w2_ref[...])
        b3s[...] = b3_ref[...] - _rm_dot(m2_ref[...] * s2, w3_ref[...])

    xb = x_ref[...].astype(jnp.bfloat16)
    h = jnp.maximum(_rm_dot(xb, w1s[...]) + b1s[...], 0.0)
    g = jnp.maximum(_rm_dot(h.astype(jnp.bfloat16), w2s[...]) + b2s[...], 0.0)
    out_ref[...] = _rm_dot(g.astype(jnp.bfloat16), w3s[...]) + b3s[...]


@functools.partial(jax.jit, static_argnames=("interpret",))
def kernel(x, W1, b1, W2, b2, W3, b3, bn0_mean, bn0_var, bn1_mean, bn1_var,
           bn2_mean, bn2_var, interpret=False):
    n = x.shape[0]
    grid = (n // _BLOCK,)

    def row_blk(i):
        return (i, 0)

    def const_blk(i):
        return (0, 0)

    full = lambda shape: pl.BlockSpec(shape, const_blk)

    return pl.pallas_call(
        _mlp_block,
        grid=grid,
        in_specs=[
            pl.BlockSpec((_BLOCK, _D_IN), row_blk),
            full((_H1, _D_IN)),
            full((1, _H1)),
            full((_H2, _H1)),
            full((1, _H2)),
            full((_D_OUT, _H2)),
            full((1, _D_OUT)),
            full((1, _D_IN)),
            full((1, _D_IN)),
            full((1, _H1)),
            full((1, _H1)),
            full((1, _H2)),
            full((1, _H2)),
        ],
        out_specs=pl.BlockSpec((_BLOCK, _D_OUT), row_blk),
        out_shape=jax.ShapeDtypeStruct((n, _D_OUT), jnp.float32),
        scratch_shapes=[
            pltpu.VMEM((_H1, _D_IN), jnp.bfloat16),
            pltpu.VMEM((_H2, _H1), jnp.bfloat16),
            pltpu.VMEM((_D_OUT, _H2), jnp.bfloat16),
            pltpu.VMEM((1, _H1), jnp.float32),
            pltpu.VMEM((1, _H2), jnp.float32),
            pltpu.VMEM((1, _D_OUT), jnp.float32),
        ],
        interpret=interpret,
    )(x, W1, b1.reshape(1, -1), W2, b2.reshape(1, -1), W3,
      b3.reshape(1, -1), bn0_mean.reshape(1, -1), bn0_var.reshape(1, -1),
      bn1_mean.reshape(1, -1), bn1_var.reshape(1, -1),
      bn2_mean.reshape(1, -1), bn2_var.reshape(1, -1))
